# baseline (speedup 1.0000x reference)
import functools

import jax
import jax.numpy as jnp
from jax import lax
from jax.experimental import pallas as pl
from jax.experimental.pallas import tpu as pltpu
from jax.experimental.pallas import tpu_sc as plsc

NC = 2
NS = 16
L = 16


def _leaky(v):
    return jnp.maximum(v, 0.01 * v)


def _pre_body(x_ref, posp_ref, wh1_ref, bh1_ref, wh2_ref, bh2_ref,
              wfx_ref, bf_ref, xf_ref, q_ref):
    xb = x_ref[...]
    h = _leaky(jnp.dot(xb, wh1_ref[...], preferred_element_type=jnp.float32)
               + bh1_ref[...])
    delta = jnp.tanh(jnp.dot(h, wh2_ref[...], preferred_element_type=jnp.float32)
                     + bh2_ref[...])
    q_ref[...] = posp_ref[...] - delta
    xf_ref[...] = (jnp.dot(xb, wfx_ref[...], preferred_element_type=jnp.float32)
                   + bf_ref[...])


def _tc_pre(x, pos_pad, wh1, bh1, wh2p, bh2p, wfx, bf, blk):
    n, c = x.shape
    grid = n // blk
    row_spec = pl.BlockSpec((blk, c), lambda i: (i, 0))
    full = lambda a: pl.BlockSpec(a.shape, lambda i: (0,) * a.ndim)
    return pl.pallas_call(
        _pre_body,
        grid=(grid,),
        in_specs=[row_spec, row_spec, full(wh1), full(bh1), full(wh2p),
                  full(bh2p), full(wfx), full(bf)],
        out_specs=[row_spec, row_spec],
        out_shape=[jax.ShapeDtypeStruct((n, c), jnp.float32),
                   jax.ShapeDtypeStruct((n, c), jnp.float32)],
    )(x, pos_pad, wh1, bh1, wh2p, bh2p, wfx, bf)


def _post_body(agg2_ref, x_ref, wg1_ref, bg1_ref, wg2_ref, bg2_ref, out_ref):
    agg = agg2_ref[0] + agg2_ref[1]
    g = _leaky(jnp.dot(agg, wg1_ref[...], preferred_element_type=jnp.float32)
               + bg1_ref[...])
    out_ref[...] = (jnp.dot(g, wg2_ref[...], preferred_element_type=jnp.float32)
                    + bg2_ref[...] + x_ref[...])


def _tc_post(agg2, x, wg1, bg1, wg2, bg2, blk):
    n, c = x.shape
    grid = n // blk
    row_spec = pl.BlockSpec((blk, c), lambda i: (i, 0))
    full = lambda a: pl.BlockSpec(a.shape, lambda i: (0,) * a.ndim)
    return pl.pallas_call(
        _post_body,
        grid=(grid,),
        in_specs=[pl.BlockSpec((2, blk, c), lambda i: (0, i, 0)),
                  row_spec, full(wg1), full(bg1), full(wg2), full(bg2)],
        out_specs=row_spec,
        out_shape=jax.ShapeDtypeStruct((n, c), jnp.float32),
    )(agg2, x, wg1, bg1, wg2, bg2)


def _make_sc_edge(n, c, e, chunks, b):
    nw = NC * NS
    rows_per_sub = n // NS
    zrows = 25
    cg = c // L
    pqw = 16

    mesh = plsc.VectorSubcoreMesh(core_axis_name="c", subcore_axis_name="s",
                                  num_cores=NC, num_subcores=NS)

    @functools.partial(
        pl.kernel,
        out_type=jax.ShapeDtypeStruct((NC, n, c), jnp.float32),
        mesh=mesh,
        scratch_types=[
            pltpu.VMEM((3, c), jnp.float32),
            pltpu.VMEM((4, b), jnp.int32),
            pltpu.VMEM((4, b), jnp.int32),
            pltpu.VMEM((2, b), jnp.int32),
            pltpu.VMEM((b, c + pqw), jnp.float32),
            pltpu.VMEM((b, c + pqw), jnp.float32),
            pltpu.VMEM((b, c), jnp.float32),
            pltpu.VMEM((b, c), jnp.float32),
            pltpu.VMEM((1, b), jnp.int32),
            pltpu.VMEM((1, b), jnp.int32),
            pltpu.VMEM((b, pqw), jnp.float32),
            pltpu.VMEM((b, pqw), jnp.float32),
            pltpu.VMEM((zrows, c), jnp.float32),
            pltpu.VMEM_SHARED((n, c), jnp.float32),
            pltpu.SemaphoreType.DMA,
            pltpu.SemaphoreType.DMA,
            pltpu.SemaphoreType.DMA,
            pltpu.SemaphoreType.DMA,
            pltpu.SemaphoreType.DMA,
        ],
        compiler_params=pltpu.CompilerParams(use_tc_tiling_on_sc=False,
                                             needs_layout_passes=False),
    )
    def sc_edge(xfp_hbm, a_hbm, ei_hbm, qt_hbm, out_hbm,
                a_ref, icur, inxt, ia, rows_a, rows_b, msg_a, msg_b,
                da, db, pqd_a, pqd_b, zbuf, aggsh,
                sem_a, sem_b, isem, ssem_a, ssem_b):
        ci = lax.axis_index("c")
        si = lax.axis_index("s")
        wid = ci * NS + si

        def zero_zbuf(i, _):
            z = jnp.zeros((L,), jnp.float32)
            for j in range(cg):
                zbuf[i, pl.ds(j * L, L)] = z
            return 0
        lax.fori_loop(0, zrows, zero_zbuf, 0)
        zbase = si * rows_per_sub

        def zero_agg(k, _):
            pltpu.sync_copy(zbuf, aggsh.at[pl.ds(zbase + k * zrows, zrows)])
            return 0
        lax.fori_loop(0, rows_per_sub // zrows, zero_agg, 0)

        pltpu.sync_copy(a_hbm, a_ref)
        plsc.subcore_barrier()

        a_vecs = [[a_ref[comp, pl.ds(j * L, L)] for j in range(cg)]
                  for comp in range(3)]

        def fire(sidx, didx, rows, pqd, sem):
            pltpu.async_copy(xfp_hbm.at[sidx], rows, sem)
            pltpu.async_copy(qt_hbm.at[didx], pqd, sem)

        def drain(rows, pqd, sem):
            pltpu.make_async_copy(xfp_hbm.at[icur.at[0]], rows, sem).wait()
            pltpu.make_async_copy(qt_hbm.at[icur.at[1]], pqd, sem).wait()

        def fetch_pair(p, sem):
            pltpu.async_copy(ei_hbm.at[wid, p], inxt, sem)

        def wait_pair(sem):
            pltpu.make_async_copy(ei_hbm.at[wid, 0], inxt, sem).wait()

        def rotate_idx():
            for r in range(4):
                for g in range(b // L):
                    sl = pl.ds(g * L, L)
                    icur[r, sl] = inxt[r, sl]

        def copy_ia():
            for r in range(2):
                for g in range(b // L):
                    sl = pl.ds(g * L, L)
                    ia[r, sl] = inxt[r, sl]

        def copy_row(dst, src_row):
            for g in range(b // L):
                sl = pl.ds(g * L, L)
                dst[0, sl] = icur[src_row, sl]

        def zero_rows(buf):
            z = jnp.zeros((L,), jnp.float32)
            def zb(i, _):
                for j in range(cg):
                    buf[i, pl.ds(j * L, L)] = z
                return 0
            lax.fori_loop(0, b, zb, 0)

        def compute(rows, msg, pqd):
            def group_body(g, _):
                base = g * L
                rowsel = base + lax.iota(jnp.int32, L)
                rv = []
                for comp in range(3):
                    p = plsc.load_gather(
                        rows, [rowsel, jnp.full((L,), c + comp, jnp.int32)])
                    q = plsc.load_gather(
                        pqd, [rowsel, jnp.full((L,), comp, jnp.int32)])
                    rv.append(p - q)
                for k in range(L):
                    r0 = rv[0][k]
                    r1 = rv[1][k]
                    r2 = rv[2][k]
                    for j in range(cg):
                        cs = pl.ds(j * L, L)
                        t = (rows[base + k, cs] + r0 * a_vecs[0][j]
                             + r1 * a_vecs[1][j] + r2 * a_vecs[2][j])
                        msg[base + k, cs] = jnp.maximum(t, 0.01 * t)
                return 0
            lax.fori_loop(0, b // L, group_body, 0)

        def scatter(msg, didx, ssem):
            del ssem
            pltpu.sync_copy(msg, aggsh.at[didx.at[0]], add=True)

        def drain_sc(msg, ssem):
            del msg, ssem

        buf_a = (rows_a, pqd_a, sem_a)
        buf_b = (rows_b, pqd_b, sem_b)
        npairs = (chunks + 1) // 2

        pltpu.async_copy(ei_hbm.at[wid, 0], inxt, isem)
        wait_pair(isem)
        rotate_idx()
        copy_ia()
        fire(ia.at[0], ia.at[1], *buf_a)
        fetch_pair(1, isem)

        def pair_body(i, _):
            fire(icur.at[2], icur.at[3], *buf_b)
            drain(*buf_a)
            drain_sc(msg_a, ssem_a)
            compute(rows_a, msg_a, pqd_a)
            copy_row(da, 1)
            scatter(msg_a, da, ssem_a)
            wait_pair(isem)
            copy_ia()
            fire(ia.at[0], ia.at[1], *buf_a)
            drain(*buf_b)
            drain_sc(msg_b, ssem_b)
            compute(rows_b, msg_b, pqd_b)
            copy_row(db, 3)
            scatter(msg_b, db, ssem_b)
            rotate_idx()
            @pl.when(i + 2 < npairs)
            def _():
                fetch_pair(i + 2, isem)
            return 0

        lax.fori_loop(0, chunks // 2, pair_body, 0)
        if chunks % 2:
            drain(*buf_a)
            drain_sc(msg_a, ssem_a)
            compute(rows_a, msg_a, pqd_a)
            copy_row(da, 1)
            scatter(msg_a, da, ssem_a)
        drain_sc(msg_a, ssem_a)
        drain_sc(msg_b, ssem_b)
        plsc.subcore_barrier()

        half = n // 2
        @pl.when(si == 0)
        def _dump_lo():
            pltpu.sync_copy(aggsh.at[pl.ds(0, half)],
                            out_hbm.at[ci, pl.ds(0, half)])
        @pl.when(si == 1)
        def _dump_hi():
            pltpu.sync_copy(aggsh.at[pl.ds(half, half)],
                            out_hbm.at[ci, pl.ds(half, half)])

    return sc_edge


def kernel(x, pos, edge_index, Wh1, bh1, Wh2, bh2, Wf, bf, Wg1, bg1, Wg2, bg2):
    n, c = x.shape
    e = edge_index.shape[1]
    nw = NC * NS
    e_per_w = e // nw
    b = 80
    chunks = e_per_w // b

    pos_pad = jnp.pad(pos, ((0, 0), (0, c - pos.shape[1])))
    wh2p = jnp.pad(Wh2, ((0, 0), (0, c - Wh2.shape[1])))
    bh2p = jnp.pad(bh2, (0, c - bh2.shape[0])).reshape(1, c)
    wfx = Wf[3:]
    xf, q_pad = _tc_pre(x, pos_pad, Wh1, bh1.reshape(1, c), wh2p, bh2p,
                        wfx, bf.reshape(1, c), blk=1000)

    xfp = jnp.concatenate([xf, pos, jnp.zeros((n, 13), jnp.float32)], axis=1)
    qt = jnp.pad(q_pad[:, :3], ((0, 0), (0, 13)))
    a_mat = Wf[:3]
    npairs = (chunks + 1) // 2
    ei = (edge_index.astype(jnp.int32).reshape(2, nw, chunks, b)
          .transpose(1, 2, 0, 3))
    ei = jnp.pad(ei, ((0, 0), (0, 2 * npairs - chunks), (0, 0), (0, 0)))
    ei = ei.reshape(nw, npairs, 4, b)

    agg2 = _make_sc_edge(n, c, e, chunks, b)(xfp, a_mat, ei, qt)

    return _tc_post(agg2, x, Wg1, bg1.reshape(1, c), Wg2, bg2.reshape(1, c),
                    blk=1000)


# speedup vs baseline: 2.7452x; 2.7452x over previous
import functools

import jax
import jax.numpy as jnp
from jax import lax
from jax.experimental import pallas as pl
from jax.experimental.pallas import tpu as pltpu
from jax.experimental.pallas import tpu_sc as plsc

NC = 2
NS = 16
L = 16


def _leaky(v):
    return jnp.maximum(v, 0.01 * v)


def _pre_body(x_ref, posp_ref, wh1_ref, bh1_ref, wh2_ref, bh2_ref,
              wfx_ref, bf_ref, xf_ref, q_ref):
    xb = x_ref[...]
    h = _leaky(jnp.dot(xb, wh1_ref[...], preferred_element_type=jnp.float32)
               + bh1_ref[...])
    delta = jnp.tanh(jnp.dot(h, wh2_ref[...], preferred_element_type=jnp.float32)
                     + bh2_ref[...])
    q_ref[...] = posp_ref[...] - delta
    xf_ref[...] = (jnp.dot(xb, wfx_ref[...], preferred_element_type=jnp.float32)
                   + bf_ref[...])


def _tc_pre(x, pos_pad, wh1, bh1, wh2p, bh2p, wfx, bf, blk):
    n, c = x.shape
    grid = n // blk
    row_spec = pl.BlockSpec((blk, c), lambda i: (i, 0))
    full = lambda a: pl.BlockSpec(a.shape, lambda i: (0,) * a.ndim)
    return pl.pallas_call(
        _pre_body,
        grid=(grid,),
        in_specs=[row_spec, row_spec, full(wh1), full(bh1), full(wh2p),
                  full(bh2p), full(wfx), full(bf)],
        out_specs=[row_spec, row_spec],
        out_shape=[jax.ShapeDtypeStruct((n, c), jnp.float32),
                   jax.ShapeDtypeStruct((n, c), jnp.float32)],
    )(x, pos_pad, wh1, bh1, wh2p, bh2p, wfx, bf)


def _post_body(agg2_ref, x_ref, wg1_ref, bg1_ref, wg2_ref, bg2_ref, out_ref):
    agg = agg2_ref[0] + agg2_ref[1]
    g = _leaky(jnp.dot(agg, wg1_ref[...], preferred_element_type=jnp.float32)
               + bg1_ref[...])
    out_ref[...] = (jnp.dot(g, wg2_ref[...], preferred_element_type=jnp.float32)
                    + bg2_ref[...] + x_ref[...])


def _tc_post(agg2, x, wg1, bg1, wg2, bg2, blk):
    n, c = x.shape
    grid = n // blk
    row_spec = pl.BlockSpec((blk, c), lambda i: (i, 0))
    full = lambda a: pl.BlockSpec(a.shape, lambda i: (0,) * a.ndim)
    return pl.pallas_call(
        _post_body,
        grid=(grid,),
        in_specs=[pl.BlockSpec((2, blk, c), lambda i: (0, i, 0)),
                  row_spec, full(wg1), full(bg1), full(wg2), full(bg2)],
        out_specs=row_spec,
        out_shape=jax.ShapeDtypeStruct((n, c), jnp.float32),
    )(agg2, x, wg1, bg1, wg2, bg2)


def _make_sc_edge(n, c, e, chunks, b):
    nw = NC * NS
    rows_per_sub = n // NS
    zrows = 25
    cg = c // L
    pqw = 16

    mesh = plsc.VectorSubcoreMesh(core_axis_name="c", subcore_axis_name="s",
                                  num_cores=NC, num_subcores=NS)

    @functools.partial(
        pl.kernel,
        out_type=jax.ShapeDtypeStruct((NC, n, c), jnp.float32),
        mesh=mesh,
        scratch_types=[
            pltpu.VMEM((3, c), jnp.float32),
            pltpu.VMEM((4, b), jnp.int32),
            pltpu.VMEM((4, b), jnp.int32),
            pltpu.VMEM((2, b), jnp.int32),
            pltpu.VMEM((b, c), jnp.float32),
            pltpu.VMEM((b, c), jnp.float32),
            pltpu.VMEM((b, pqw), jnp.float32),
            pltpu.VMEM((b, pqw), jnp.float32),
            pltpu.VMEM((b, pqw), jnp.float32),
            pltpu.VMEM((b, pqw), jnp.float32),
            pltpu.VMEM((zrows, c), jnp.float32),
            pltpu.VMEM_SHARED((n, c), jnp.float32),
            pltpu.SemaphoreType.DMA,
            pltpu.SemaphoreType.DMA,
            pltpu.SemaphoreType.DMA,
        ],
        compiler_params=pltpu.CompilerParams(use_tc_tiling_on_sc=False,
                                             needs_layout_passes=False),
    )
    def sc_edge(posq_hbm, a_hbm, ei_hbm, xf_hbm, out_hbm,
                a_ref, icur, inxt, ia, rows_a, rows_b, pqs_a, pqs_b,
                pqd_a, pqd_b, zbuf, aggsh, sem_a, sem_b, isem):
        ci = lax.axis_index("c")
        si = lax.axis_index("s")
        wid = ci * NS + si

        def zero_zbuf(i, _):
            z = jnp.zeros((L,), jnp.float32)
            for j in range(cg):
                zbuf[i, pl.ds(j * L, L)] = z
            return 0
        lax.fori_loop(0, zrows, zero_zbuf, 0)
        zbase = si * rows_per_sub

        def zero_agg(k, _):
            pltpu.sync_copy(zbuf, aggsh.at[pl.ds(zbase + k * zrows, zrows)])
            return 0
        lax.fori_loop(0, rows_per_sub // zrows, zero_agg, 0)

        pltpu.sync_copy(a_hbm, a_ref)
        plsc.subcore_barrier()

        a_vecs = [[a_ref[comp, pl.ds(j * L, L)] for j in range(cg)]
                  for comp in range(3)]

        def fire(sidx, didx, rows, pqs, pqd, sem):
            pltpu.async_copy(xf_hbm.at[sidx], rows, sem)
            pltpu.async_copy(posq_hbm.at[sidx], pqs, sem)
            pltpu.async_copy(posq_hbm.at[didx], pqd, sem)

        def drain(rows, pqs, pqd, sem):
            pltpu.make_async_copy(xf_hbm.at[icur.at[0]], rows, sem).wait()
            pltpu.make_async_copy(posq_hbm.at[icur.at[0]], pqs, sem).wait()
            pltpu.make_async_copy(posq_hbm.at[icur.at[1]], pqd, sem).wait()

        def fetch_pair(p, sem):
            pltpu.async_copy(ei_hbm.at[wid, p], inxt, sem)

        def wait_pair(sem):
            pltpu.make_async_copy(ei_hbm.at[wid, 0], inxt, sem).wait()

        def rotate_idx():
            for r in range(4):
                for g in range(b // L):
                    sl = pl.ds(g * L, L)
                    icur[r, sl] = inxt[r, sl]

        def copy_ia():
            for r in range(2):
                for g in range(b // L):
                    sl = pl.ds(g * L, L)
                    ia[r, sl] = inxt[r, sl]

        def compute(didx, rows, pqs, pqd):
            def group_body(g, _):
                base = g * L
                rowsel = base + lax.iota(jnp.int32, L)
                rv = []
                for comp in range(3):
                    p = plsc.load_gather(
                        pqs, [rowsel, jnp.full((L,), comp, jnp.int32)])
                    q = plsc.load_gather(
                        pqd, [rowsel, jnp.full((L,), comp + 3, jnp.int32)])
                    rv.append(p - q)
                for k in range(L):
                    r0 = rv[0][k]
                    r1 = rv[1][k]
                    r2 = rv[2][k]
                    for j in range(cg):
                        cs = pl.ds(j * L, L)
                        t = (rows[base + k, cs] + r0 * a_vecs[0][j]
                             + r1 * a_vecs[1][j] + r2 * a_vecs[2][j])
                        rows[base + k, cs] = jnp.maximum(t, 0.01 * t)
                return 0
            lax.fori_loop(0, b // L, group_body, 0)
            pltpu.sync_copy(rows, aggsh.at[didx], add=True)

        buf_a = (rows_a, pqs_a, pqd_a, sem_a)
        buf_b = (rows_b, pqs_b, pqd_b, sem_b)
        npairs = (chunks + 1) // 2

        pltpu.async_copy(ei_hbm.at[wid, 0], inxt, isem)
        wait_pair(isem)
        rotate_idx()
        copy_ia()
        fire(ia.at[0], ia.at[1], *buf_a)
        fetch_pair(1, isem)

        def pair_body(i, _):
            fire(icur.at[2], icur.at[3], *buf_b)
            drain(*buf_a)
            compute(icur.at[1], *buf_a[:3])
            wait_pair(isem)
            copy_ia()
            fire(ia.at[0], ia.at[1], *buf_a)
            drain(*buf_b)
            compute(icur.at[3], *buf_b[:3])
            rotate_idx()
            @pl.when(i + 2 < npairs)
            def _():
                fetch_pair(i + 2, isem)
            return 0

        lax.fori_loop(0, chunks // 2, pair_body, 0)
        if chunks % 2:
            drain(*buf_a)
            compute(icur.at[1], *buf_a[:3])
        plsc.subcore_barrier()

        half = n // 2
        @pl.when(si == 0)
        def _dump_lo():
            pltpu.sync_copy(aggsh.at[pl.ds(0, half)],
                            out_hbm.at[ci, pl.ds(0, half)])
        @pl.when(si == 1)
        def _dump_hi():
            pltpu.sync_copy(aggsh.at[pl.ds(half, half)],
                            out_hbm.at[ci, pl.ds(half, half)])

    return sc_edge


def kernel(x, pos, edge_index, Wh1, bh1, Wh2, bh2, Wf, bf, Wg1, bg1, Wg2, bg2):
    n, c = x.shape
    e = edge_index.shape[1]
    nw = NC * NS
    e_per_w = e // nw
    b = 80
    chunks = e_per_w // b

    pos_pad = jnp.pad(pos, ((0, 0), (0, c - pos.shape[1])))
    wh2p = jnp.pad(Wh2, ((0, 0), (0, c - Wh2.shape[1])))
    bh2p = jnp.pad(bh2, (0, c - bh2.shape[0])).reshape(1, c)
    wfx = Wf[3:]
    xf, q_pad = _tc_pre(x, pos_pad, Wh1, bh1.reshape(1, c), wh2p, bh2p,
                        wfx, bf.reshape(1, c), blk=1000)

    posq = jnp.concatenate(
        [pos, q_pad[:, :3], jnp.zeros((n, 10), jnp.float32)], axis=1)
    a_mat = Wf[:3]
    npairs = (chunks + 1) // 2
    ei = (edge_index.astype(jnp.int32).reshape(2, nw, chunks, b)
          .transpose(1, 2, 0, 3))
    ei = jnp.pad(ei, ((0, 0), (0, 2 * npairs - chunks), (0, 0), (0, 0)))
    ei = ei.reshape(nw, npairs, 4, b)

    agg2 = _make_sc_edge(n, c, e, chunks, b)(posq, a_mat, ei, xf)

    return _tc_post(agg2, x, Wg1, bg1.reshape(1, c), Wg2, bg2.reshape(1, c),
                    blk=1000)


# speedup vs baseline: 2.7957x; 1.0184x over previous
import functools

import jax
import jax.numpy as jnp
from jax import lax
from jax.experimental import pallas as pl
from jax.experimental.pallas import tpu as pltpu
from jax.experimental.pallas import tpu_sc as plsc

NC = 2
NS = 16
L = 16


def _leaky(v):
    return jnp.maximum(v, 0.01 * v)


def _pre_body(x_ref, posp_ref, wh1_ref, bh1_ref, wh2_ref, bh2_ref,
              wfx_ref, bf_ref, xf_ref, q_ref):
    xb = x_ref[...]
    h = _leaky(jnp.dot(xb, wh1_ref[...], preferred_element_type=jnp.float32)
               + bh1_ref[...])
    delta = jnp.tanh(jnp.dot(h, wh2_ref[...], preferred_element_type=jnp.float32)
                     + bh2_ref[...])
    q_ref[...] = posp_ref[...] - delta
    xf_ref[...] = (jnp.dot(xb, wfx_ref[...], preferred_element_type=jnp.float32)
                   + bf_ref[...])


def _tc_pre(x, pos_pad, wh1, bh1, wh2p, bh2p, wfx, bf, blk):
    n, c = x.shape
    grid = n // blk
    row_spec = pl.BlockSpec((blk, c), lambda i: (i, 0))
    full = lambda a: pl.BlockSpec(a.shape, lambda i: (0,) * a.ndim)
    return pl.pallas_call(
        _pre_body,
        grid=(grid,),
        in_specs=[row_spec, row_spec, full(wh1), full(bh1), full(wh2p),
                  full(bh2p), full(wfx), full(bf)],
        out_specs=[row_spec, row_spec],
        out_shape=[jax.ShapeDtypeStruct((n, c), jnp.float32),
                   jax.ShapeDtypeStruct((n, c), jnp.float32)],
    )(x, pos_pad, wh1, bh1, wh2p, bh2p, wfx, bf)


def _post_body(agg2_ref, x_ref, wg1_ref, bg1_ref, wg2_ref, bg2_ref, out_ref):
    agg = agg2_ref[0] + agg2_ref[1]
    g = _leaky(jnp.dot(agg, wg1_ref[...], preferred_element_type=jnp.float32)
               + bg1_ref[...])
    out_ref[...] = (jnp.dot(g, wg2_ref[...], preferred_element_type=jnp.float32)
                    + bg2_ref[...] + x_ref[...])


def _tc_post(agg2, x, wg1, bg1, wg2, bg2, blk):
    n, c = x.shape
    grid = n // blk
    row_spec = pl.BlockSpec((blk, c), lambda i: (i, 0))
    full = lambda a: pl.BlockSpec(a.shape, lambda i: (0,) * a.ndim)
    return pl.pallas_call(
        _post_body,
        grid=(grid,),
        in_specs=[pl.BlockSpec((2, blk, c), lambda i: (0, i, 0)),
                  row_spec, full(wg1), full(bg1), full(wg2), full(bg2)],
        out_specs=row_spec,
        out_shape=jax.ShapeDtypeStruct((n, c), jnp.float32),
    )(agg2, x, wg1, bg1, wg2, bg2)


def _make_sc_edge(n, c, e, chunks, b):
    nw = NC * NS
    rows_per_sub = n // NS
    zrows = 25
    cg = c // L
    pqw = 16

    mesh = plsc.VectorSubcoreMesh(core_axis_name="c", subcore_axis_name="s",
                                  num_cores=NC, num_subcores=NS)

    @functools.partial(
        pl.kernel,
        out_type=jax.ShapeDtypeStruct((NC, n, c), jnp.float32),
        mesh=mesh,
        scratch_types=[
            pltpu.VMEM((3, c), jnp.float32),
            pltpu.VMEM((4, b), jnp.int32),
            pltpu.VMEM((4, b), jnp.int32),
            pltpu.VMEM((2, b), jnp.int32),
            pltpu.VMEM((b, c), jnp.float32),
            pltpu.VMEM((b, c), jnp.float32),
            pltpu.VMEM((b, pqw), jnp.float32),
            pltpu.VMEM((b, pqw), jnp.float32),
            pltpu.VMEM((b, pqw), jnp.float32),
            pltpu.VMEM((b, pqw), jnp.float32),
            pltpu.VMEM((zrows, c), jnp.float32),
            pltpu.VMEM_SHARED((n, c), jnp.float32),
            pltpu.SemaphoreType.DMA,
            pltpu.SemaphoreType.DMA,
            pltpu.SemaphoreType.DMA,
        ],
        compiler_params=pltpu.CompilerParams(use_tc_tiling_on_sc=False,
                                             needs_layout_passes=False),
    )
    def sc_edge(posq_hbm, a_hbm, ei_hbm, xf_hbm, out_hbm,
                a_ref, icur, inxt, ia, rows_a, rows_b, pqs_a, pqs_b,
                pqd_a, pqd_b, zbuf, aggsh, sem_a, sem_b, isem):
        ci = lax.axis_index("c")
        si = lax.axis_index("s")
        wid = ci * NS + si

        def zero_zbuf(i, _):
            z = jnp.zeros((L,), jnp.float32)
            for j in range(cg):
                zbuf[i, pl.ds(j * L, L)] = z
            return 0
        lax.fori_loop(0, zrows, zero_zbuf, 0)
        zbase = si * rows_per_sub

        def zero_agg(k, _):
            pltpu.sync_copy(zbuf, aggsh.at[pl.ds(zbase + k * zrows, zrows)])
            return 0
        lax.fori_loop(0, rows_per_sub // zrows, zero_agg, 0)

        pltpu.sync_copy(a_hbm, a_ref)
        plsc.subcore_barrier()

        a_vecs = [[a_ref[comp, pl.ds(j * L, L)] for j in range(cg)]
                  for comp in range(3)]

        def fire(sidx, didx, rows, pqs, pqd, sem):
            pltpu.async_copy(xf_hbm.at[sidx], rows, sem)
            pltpu.async_copy(posq_hbm.at[sidx], pqs, sem)
            pltpu.async_copy(posq_hbm.at[didx], pqd, sem)

        def drain(rows, pqs, pqd, sem):
            pltpu.make_async_copy(xf_hbm.at[icur.at[0]], rows, sem).wait()
            pltpu.make_async_copy(posq_hbm.at[icur.at[0]], pqs, sem).wait()
            pltpu.make_async_copy(posq_hbm.at[icur.at[1]], pqd, sem).wait()

        def fetch_pair(p, sem):
            pltpu.async_copy(ei_hbm.at[wid, p], inxt, sem)

        def wait_pair(sem):
            pltpu.make_async_copy(ei_hbm.at[wid, 0], inxt, sem).wait()

        def rotate_idx():
            for r in range(4):
                for g in range(b // L):
                    sl = pl.ds(g * L, L)
                    icur[r, sl] = inxt[r, sl]

        def copy_ia():
            for r in range(2):
                for g in range(b // L):
                    sl = pl.ds(g * L, L)
                    ia[r, sl] = inxt[r, sl]

        def compute(didx, rows, pqs, pqd):
            def group_body(g, _):
                base = g * L
                rowsel = base + lax.iota(jnp.int32, L)
                rv = []
                for comp in range(3):
                    p = plsc.load_gather(
                        pqs, [rowsel, jnp.full((L,), comp, jnp.int32)])
                    q = plsc.load_gather(
                        pqd, [rowsel, jnp.full((L,), comp + 3, jnp.int32)])
                    rv.append(p - q)
                for k in range(L):
                    r0 = rv[0][k]
                    r1 = rv[1][k]
                    r2 = rv[2][k]
                    for j in range(cg):
                        cs = pl.ds(j * L, L)
                        t = (rows[base + k, cs] + r0 * a_vecs[0][j]
                             + r1 * a_vecs[1][j] + r2 * a_vecs[2][j])
                        rows[base + k, cs] = jnp.maximum(t, 0.01 * t)
                return 0
            lax.fori_loop(0, b // L, group_body, 0)
            pltpu.sync_copy(rows, aggsh.at[didx], add=True)

        buf_a = (rows_a, pqs_a, pqd_a, sem_a)
        buf_b = (rows_b, pqs_b, pqd_b, sem_b)
        npairs = (chunks + 1) // 2

        pltpu.async_copy(ei_hbm.at[wid, 0], inxt, isem)
        wait_pair(isem)
        rotate_idx()
        copy_ia()
        fire(ia.at[0], ia.at[1], *buf_a)
        fetch_pair(1, isem)

        def pair_body(i, _):
            fire(icur.at[2], icur.at[3], *buf_b)
            drain(*buf_a)
            compute(icur.at[1], *buf_a[:3])
            wait_pair(isem)
            copy_ia()
            fire(ia.at[0], ia.at[1], *buf_a)
            drain(*buf_b)
            compute(icur.at[3], *buf_b[:3])
            rotate_idx()
            @pl.when(i + 2 < npairs)
            def _():
                fetch_pair(i + 2, isem)
            return 0

        lax.fori_loop(0, chunks // 2, pair_body, 0)
        if chunks % 2:
            drain(*buf_a)
            compute(icur.at[1], *buf_a[:3])
        plsc.subcore_barrier()

        half = n // 2
        @pl.when(si == 0)
        def _dump_lo():
            pltpu.sync_copy(aggsh.at[pl.ds(0, half)],
                            out_hbm.at[ci, pl.ds(0, half)])
        @pl.when(si == 1)
        def _dump_hi():
            pltpu.sync_copy(aggsh.at[pl.ds(half, half)],
                            out_hbm.at[ci, pl.ds(half, half)])

    return sc_edge


def kernel(x, pos, edge_index, Wh1, bh1, Wh2, bh2, Wf, bf, Wg1, bg1, Wg2, bg2):
    n, c = x.shape
    e = edge_index.shape[1]
    nw = NC * NS
    e_per_w = e // nw
    b = 80
    chunks = e_per_w // b

    pos_pad = jnp.pad(pos, ((0, 0), (0, c - pos.shape[1])))
    wh2p = jnp.pad(Wh2, ((0, 0), (0, c - Wh2.shape[1])))
    bh2p = jnp.pad(bh2, (0, c - bh2.shape[0])).reshape(1, c)
    wfx = Wf[3:]
    xf, q_pad = _tc_pre(x, pos_pad, Wh1, bh1.reshape(1, c), wh2p, bh2p,
                        wfx, bf.reshape(1, c), blk=2000)

    posq = jnp.concatenate(
        [pos, q_pad[:, :3], jnp.zeros((n, 10), jnp.float32)], axis=1)
    a_mat = Wf[:3]
    npairs = (chunks + 1) // 2
    ei = (edge_index.astype(jnp.int32).reshape(2, nw, chunks, b)
          .transpose(1, 2, 0, 3))
    ei = jnp.pad(ei, ((0, 0), (0, 2 * npairs - chunks), (0, 0), (0, 0)))
    ei = ei.reshape(nw, npairs, 4, b)

    agg2 = _make_sc_edge(n, c, e, chunks, b)(posq, a_mat, ei, xf)

    return _tc_post(agg2, x, Wg1, bg1.reshape(1, c), Wg2, bg2.reshape(1, c),
                    blk=2000)


# speedup vs baseline: 2.8148x; 1.0068x over previous
import functools

import jax
import jax.numpy as jnp
from jax import lax
from jax.experimental import pallas as pl
from jax.experimental.pallas import tpu as pltpu
from jax.experimental.pallas import tpu_sc as plsc

NC = 2
NS = 16
L = 16


def _leaky(v):
    return jnp.maximum(v, 0.01 * v)


def _pre_body(x_ref, posp_ref, wh1_ref, bh1_ref, wh2_ref, bh2_ref,
              wfx_ref, bf_ref, xf_ref, q_ref):
    xb = x_ref[...]
    h = _leaky(jnp.dot(xb, wh1_ref[...], preferred_element_type=jnp.float32)
               + bh1_ref[...])
    delta = jnp.tanh(jnp.dot(h, wh2_ref[...], preferred_element_type=jnp.float32)
                     + bh2_ref[...])
    q_ref[...] = posp_ref[...] - delta
    xf_ref[...] = (jnp.dot(xb, wfx_ref[...], preferred_element_type=jnp.float32)
                   + bf_ref[...])


def _tc_pre(x, pos_pad, wh1, bh1, wh2p, bh2p, wfx, bf, blk):
    n, c = x.shape
    grid = n // blk
    row_spec = pl.BlockSpec((blk, c), lambda i: (i, 0))
    full = lambda a: pl.BlockSpec(a.shape, lambda i: (0,) * a.ndim)
    return pl.pallas_call(
        _pre_body,
        grid=(grid,),
        in_specs=[row_spec, row_spec, full(wh1), full(bh1), full(wh2p),
                  full(bh2p), full(wfx), full(bf)],
        out_specs=[row_spec, row_spec],
        out_shape=[jax.ShapeDtypeStruct((n, c), jnp.float32),
                   jax.ShapeDtypeStruct((n, c), jnp.float32)],
    )(x, pos_pad, wh1, bh1, wh2p, bh2p, wfx, bf)


def _post_body(agg2_ref, x_ref, wg1_ref, bg1_ref, wg2_ref, bg2_ref, out_ref):
    agg = agg2_ref[0] + agg2_ref[1]
    g = _leaky(jnp.dot(agg, wg1_ref[...], preferred_element_type=jnp.float32)
               + bg1_ref[...])
    out_ref[...] = (jnp.dot(g, wg2_ref[...], preferred_element_type=jnp.float32)
                    + bg2_ref[...] + x_ref[...])


def _tc_post(agg2, x, wg1, bg1, wg2, bg2, blk):
    n, c = x.shape
    grid = n // blk
    row_spec = pl.BlockSpec((blk, c), lambda i: (i, 0))
    full = lambda a: pl.BlockSpec(a.shape, lambda i: (0,) * a.ndim)
    return pl.pallas_call(
        _post_body,
        grid=(grid,),
        in_specs=[pl.BlockSpec((2, blk, c), lambda i: (0, i, 0)),
                  row_spec, full(wg1), full(bg1), full(wg2), full(bg2)],
        out_specs=row_spec,
        out_shape=jax.ShapeDtypeStruct((n, c), jnp.float32),
    )(agg2, x, wg1, bg1, wg2, bg2)


def _make_sc_edge(n, c, e, chunks, b):
    nw = NC * NS
    rows_per_sub = n // NS
    zrows = 25
    cg = c // L
    pqw = 16

    mesh = plsc.VectorSubcoreMesh(core_axis_name="c", subcore_axis_name="s",
                                  num_cores=NC, num_subcores=NS)

    @functools.partial(
        pl.kernel,
        out_type=jax.ShapeDtypeStruct((NC, n, c), jnp.float32),
        mesh=mesh,
        scratch_types=[
            pltpu.VMEM((3, c), jnp.float32),
            pltpu.VMEM((4, b), jnp.int32),
            pltpu.VMEM((4, b), jnp.int32),
            pltpu.VMEM((2, b), jnp.int32),
            pltpu.VMEM((b, c), jnp.float32),
            pltpu.VMEM((b, c), jnp.float32),
            pltpu.VMEM((b, pqw), jnp.float32),
            pltpu.VMEM((b, pqw), jnp.float32),
            pltpu.VMEM((b, pqw), jnp.float32),
            pltpu.VMEM((b, pqw), jnp.float32),
            pltpu.VMEM((zrows, c), jnp.float32),
            pltpu.VMEM_SHARED((n, c), jnp.float32),
            pltpu.SemaphoreType.DMA,
            pltpu.SemaphoreType.DMA,
            pltpu.SemaphoreType.DMA,
        ],
        compiler_params=pltpu.CompilerParams(use_tc_tiling_on_sc=False,
                                             needs_layout_passes=False),
    )
    def sc_edge(posq_hbm, a_hbm, ei_hbm, xf_hbm, out_hbm,
                a_ref, icur, inxt, ia, rows_a, rows_b, pqs_a, pqs_b,
                pqd_a, pqd_b, zbuf, aggsh, sem_a, sem_b, isem):
        ci = lax.axis_index("c")
        si = lax.axis_index("s")
        wid = ci * NS + si

        def zero_zbuf(i, _):
            z = jnp.zeros((L,), jnp.float32)
            for j in range(cg):
                zbuf[i, pl.ds(j * L, L)] = z
            return 0
        lax.fori_loop(0, zrows, zero_zbuf, 0)
        zbase = si * rows_per_sub

        def zero_agg(k, _):
            pltpu.sync_copy(zbuf, aggsh.at[pl.ds(zbase + k * zrows, zrows)])
            return 0
        lax.fori_loop(0, rows_per_sub // zrows, zero_agg, 0)

        pltpu.sync_copy(a_hbm, a_ref)
        plsc.subcore_barrier()

        a_vecs = [[a_ref[comp, pl.ds(j * L, L)] for j in range(cg)]
                  for comp in range(3)]

        def fire(sidx, didx, rows, pqs, pqd, sem):
            pltpu.async_copy(xf_hbm.at[sidx], rows, sem)
            pltpu.async_copy(posq_hbm.at[sidx], pqs, sem)
            pltpu.async_copy(posq_hbm.at[didx], pqd, sem)

        def drain(rows, pqs, pqd, sem):
            pltpu.make_async_copy(xf_hbm.at[icur.at[0]], rows, sem).wait()
            pltpu.make_async_copy(posq_hbm.at[icur.at[0]], pqs, sem).wait()
            pltpu.make_async_copy(posq_hbm.at[icur.at[1]], pqd, sem).wait()

        def fetch_pair(p, sem):
            pltpu.async_copy(ei_hbm.at[wid, p], inxt, sem)

        def wait_pair(sem):
            pltpu.make_async_copy(ei_hbm.at[wid, 0], inxt, sem).wait()

        def rotate_idx():
            for r in range(4):
                for g in range(b // L):
                    sl = pl.ds(g * L, L)
                    icur[r, sl] = inxt[r, sl]

        def copy_ia():
            for r in range(2):
                for g in range(b // L):
                    sl = pl.ds(g * L, L)
                    ia[r, sl] = inxt[r, sl]

        def compute(didx, rows, pqs, pqd):
            def group_body(g, _):
                base = g * L
                rowsel = base + lax.iota(jnp.int32, L)
                rv = []
                for comp in range(3):
                    p = plsc.load_gather(
                        pqs, [rowsel, jnp.full((L,), comp, jnp.int32)])
                    q = plsc.load_gather(
                        pqd, [rowsel, jnp.full((L,), comp + 3, jnp.int32)])
                    rv.append(p - q)
                for k in range(L):
                    r0 = rv[0][k]
                    r1 = rv[1][k]
                    r2 = rv[2][k]
                    for j in range(cg):
                        cs = pl.ds(j * L, L)
                        t = (rows[base + k, cs] + r0 * a_vecs[0][j]
                             + r1 * a_vecs[1][j] + r2 * a_vecs[2][j])
                        rows[base + k, cs] = jnp.maximum(t, 0.01 * t)
                return 0
            lax.fori_loop(0, b // L, group_body, 0)
            pltpu.sync_copy(rows, aggsh.at[didx], add=True)

        buf_a = (rows_a, pqs_a, pqd_a, sem_a)
        buf_b = (rows_b, pqs_b, pqd_b, sem_b)
        npairs = (chunks + 1) // 2

        pltpu.async_copy(ei_hbm.at[wid, 0], inxt, isem)
        wait_pair(isem)
        rotate_idx()
        copy_ia()
        fire(ia.at[0], ia.at[1], *buf_a)
        fetch_pair(1, isem)

        def pair_body(i, _):
            fire(icur.at[2], icur.at[3], *buf_b)
            drain(*buf_a)
            compute(icur.at[1], *buf_a[:3])
            wait_pair(isem)
            copy_ia()
            fire(ia.at[0], ia.at[1], *buf_a)
            drain(*buf_b)
            compute(icur.at[3], *buf_b[:3])
            rotate_idx()
            @pl.when(i + 2 < npairs)
            def _():
                fetch_pair(i + 2, isem)
            return 0

        lax.fori_loop(0, chunks // 2, pair_body, 0)
        if chunks % 2:
            drain(*buf_a)
            compute(icur.at[1], *buf_a[:3])
        plsc.subcore_barrier()

        half = n // 2
        @pl.when(si == 0)
        def _dump_lo():
            pltpu.sync_copy(aggsh.at[pl.ds(0, half)],
                            out_hbm.at[ci, pl.ds(0, half)])
        @pl.when(si == 1)
        def _dump_hi():
            pltpu.sync_copy(aggsh.at[pl.ds(half, half)],
                            out_hbm.at[ci, pl.ds(half, half)])

    return sc_edge


def kernel(x, pos, edge_index, Wh1, bh1, Wh2, bh2, Wf, bf, Wg1, bg1, Wg2, bg2):
    n, c = x.shape
    e = edge_index.shape[1]
    nw = NC * NS
    e_per_w = e // nw
    b = 80
    chunks = e_per_w // b

    pos_pad = jnp.pad(pos, ((0, 0), (0, c - pos.shape[1])))
    wh2p = jnp.pad(Wh2, ((0, 0), (0, c - Wh2.shape[1])))
    bh2p = jnp.pad(bh2, (0, c - bh2.shape[0])).reshape(1, c)
    wfx = Wf[3:]
    xf, q_pad = _tc_pre(x, pos_pad, Wh1, bh1.reshape(1, c), wh2p, bh2p,
                        wfx, bf.reshape(1, c), blk=5000)

    posq = jnp.concatenate(
        [pos, q_pad[:, :3], jnp.zeros((n, 10), jnp.float32)], axis=1)
    a_mat = Wf[:3]
    npairs = (chunks + 1) // 2
    ei = (edge_index.astype(jnp.int32).reshape(2, nw, chunks, b)
          .transpose(1, 2, 0, 3))
    ei = jnp.pad(ei, ((0, 0), (0, 2 * npairs - chunks), (0, 0), (0, 0)))
    ei = ei.reshape(nw, npairs, 4, b)

    agg2 = _make_sc_edge(n, c, e, chunks, b)(posq, a_mat, ei, xf)

    return _tc_post(agg2, x, Wg1, bg1.reshape(1, c), Wg2, bg2.reshape(1, c),
                    blk=5000)
